# Initial kernel scaffold; baseline (speedup 1.0000x reference)
import jax
import jax.numpy as jnp
from jax.experimental import pallas as pl


def kernel(x, W1, b1, g1, bb1, W2, b2, g2, bb2, W3, b3, g3, bb3, W4, b4, g4, bb4):
    raise NotImplementedError("write your pallas kernel here")


# speedup vs baseline: 6.0834x; 6.0834x over previous
import functools

import jax
import jax.numpy as jnp
from jax import lax
from jax.experimental import pallas as pl
from jax.experimental.pallas import tpu as pltpu
from jax.experimental.pallas import tpu_sc as plsc

KNN = 20
EPS = 1e-5
_PAD = 16
_KPAD = 32
_RB = 256
_PB = 2560
_NSC_WORKERS = 32


def _topk_body(xr, xf, out_ref):
    b = pl.program_id(0)
    n = xf.shape[1]
    xblk = xr[0]
    xfull = xf[0]
    g = lax.dot_general(xblk, xfull, (((1,), (1,)), ((), ())),
                        preferred_element_type=jnp.float32)
    xxr = jnp.sum(xblk * xblk, axis=1, keepdims=True)
    xxf = jnp.sum(xfull * xfull, axis=1, keepdims=True)
    inner = -2.0 * g
    pd = (-xxr) - inner
    pd = pd - xxf.T
    iota = lax.broadcasted_iota(jnp.int32, pd.shape, 1)
    for r in range(KNN):
        m = jnp.max(pd, axis=1, keepdims=True)
        eq = pd == m
        col = jnp.min(jnp.where(eq, iota, n), axis=1)
        out_ref[0, :, r] = col + b * n
        pd = jnp.where(iota == col[:, None], -3e38, pd)


def _topk(xp):
    bsz, n, _ = xp.shape
    rb = min(_RB, n)
    return pl.pallas_call(
        _topk_body,
        grid=(bsz, n // rb),
        in_specs=[
            pl.BlockSpec((1, rb, _PAD), lambda b, r: (b, r, 0)),
            pl.BlockSpec((1, n, _PAD), lambda b, r: (b, 0, 0)),
        ],
        out_specs=pl.BlockSpec((1, rb, _KPAD), lambda b, r: (b, r, 0)),
        out_shape=jax.ShapeDtypeStruct((bsz, n, _KPAD), jnp.int32),
    )(xp, xp)


def _pick_chunk(rows_per_w):
    max_rows = (256 * 1024) // (_PAD * 4)
    ch = rows_per_w
    while ch > max_rows or ch % 8:
        for div in range(2, rows_per_w + 1):
            if rows_per_w % div == 0 and rows_per_w // div <= max_rows:
                ch = rows_per_w // div
                break
        else:
            ch = rows_per_w
            break
        break
    return ch


def _sc_gather(table, idx_all):
    nidx = idx_all.shape[0]
    rows_per_w = nidx // _NSC_WORKERS
    ch = _pick_chunk(rows_per_w)
    chunks = rows_per_w // ch
    mesh = plsc.VectorSubcoreMesh(core_axis_name="c", subcore_axis_name="s")

    @functools.partial(
        pl.kernel,
        out_type=jax.ShapeDtypeStruct((nidx, _PAD), jnp.float32),
        mesh=mesh,
        compiler_params=pltpu.CompilerParams(use_tc_tiling_on_sc=False),
        scratch_types=[
            pltpu.VMEM((ch,), jnp.int32),
            pltpu.VMEM((ch, _PAD), jnp.float32),
            pltpu.SemaphoreType.DMA,
        ],
    )
    def _gather_kernel(table_hbm, idx_hbm, out_hbm, idx_v, rows_v, sem):
        wid = lax.axis_index("s") * 2 + lax.axis_index("c")
        for j in range(chunks):
            base = wid * rows_per_w + j * ch
            pltpu.sync_copy(idx_hbm.at[pl.ds(base, ch)], idx_v)
            pltpu.async_copy(table_hbm.at[idx_v], rows_v, sem).wait()
            pltpu.sync_copy(rows_v, out_hbm.at[pl.ds(base, ch)])

    return _gather_kernel(table, idx_all)


def _affine(s, stats_ref, g_ref, bb_ref, npos):
    mean = stats_ref[0:1, :] / npos
    var = stats_ref[1:2, :] / npos - mean * mean
    return (s - mean) / jnp.sqrt(var + EPS) * g_ref[0:1, :] + bb_ref[0:1, :]


def _acc_stats(stats_ref, s):
    @pl.when(pl.program_id(0) == 0)
    def _():
        stats_ref[...] = jnp.zeros_like(stats_ref)

    stats_ref[0:1, :] += jnp.sum(s, axis=0, keepdims=True)
    stats_ref[1:2, :] += jnp.sum(s * s, axis=0, keepdims=True)


def _mm(a, w):
    return lax.dot_general(a, w, (((1,), (1,)), ((), ())),
                           preferred_element_type=jnp.float32)


def _make_pass_body(stage, npos):
    def body(*refs):
        ctr, nb, w1c, w1n, b1 = refs[:5]
        c = ctr[...]
        s = _mm(c, w1c[...]) + _mm(nb[...] - c, w1n[...]) + b1[0:1, :]
        i = 5
        for params in range(2, stage + 1):
            st, g, bb, w, bvec = refs[i:i + 5]
            i += 5
            h = jnp.maximum(_affine(s, st, g, bb, npos), 0.0)
            s = _mm(h, w[...]) + bvec[0:1, :]
        if stage < 4:
            _acc_stats(refs[-1], s)
        else:
            stats_ref, m_ref = refs[-2], refs[-1]
            _acc_stats(stats_ref, s)
            r = s.shape[0] // KNN
            m_ref[...] = jnp.max(s.reshape(r, KNN, s.shape[1]), axis=1)
    return body


def _run_pass(stage, nb2, wlist, npos):
    nblk = npos // _PB
    cdims = [64, 64, 128, 128]
    cs = cdims[stage - 1]
    full = lambda a: pl.BlockSpec(a.shape, lambda i: tuple(0 for _ in a.shape))
    in_specs = [
        pl.BlockSpec((_PB, _PAD), lambda i: (i, 0)),
        pl.BlockSpec((_PB, _PAD), lambda i: (i + nblk, 0)),
    ] + [full(w) for w in wlist]
    out_specs = pl.BlockSpec((8, cs), lambda i: (0, 0))
    out_shape = jax.ShapeDtypeStruct((8, cs), jnp.float32)
    if stage == 4:
        rpb = _PB // KNN
        out_specs = [out_specs, pl.BlockSpec((rpb, cs), lambda i: (i, 0))]
        out_shape = [out_shape,
                     jax.ShapeDtypeStruct((npos // KNN, cs), jnp.float32)]
    return pl.pallas_call(
        _make_pass_body(stage, float(npos)),
        grid=(nblk,),
        in_specs=in_specs,
        out_specs=out_specs,
        out_shape=out_shape,
    )(nb2, nb2, *wlist)


def _final_body(m_ref, st_ref, g_ref, bb_ref, out_ref, *, npos):
    val = _affine(m_ref[...], st_ref, g_ref, bb_ref, npos)
    out_ref[0] = val.T


def _finalize(m, stats4, g4, bb4, bsz, n, npos):
    c = m.shape[1]
    nj = n // 128
    return pl.pallas_call(
        functools.partial(_final_body, npos=float(npos)),
        grid=(bsz, nj),
        in_specs=[
            pl.BlockSpec((128, c), lambda b, j: (b * nj + j, 0)),
            pl.BlockSpec((8, c), lambda b, j: (0, 0)),
            pl.BlockSpec((1, c), lambda b, j: (0, 0)),
            pl.BlockSpec((1, c), lambda b, j: (0, 0)),
        ],
        out_specs=pl.BlockSpec((1, c, 128), lambda b, j: (b, 0, j)),
        out_shape=jax.ShapeDtypeStruct((bsz, c, n), jnp.float32),
    )(m, stats4, g4, bb4)


def kernel(x, W1, b1, g1, bb1, W2, b2, g2, bb2, W3, b3, g3, bb3,
           W4, b4, g4, bb4):
    bsz, n, _ = x.shape
    npos = bsz * n * KNN
    xp = jnp.pad(x, ((0, 0), (0, 0), (0, _PAD - 3)))
    idx = _topk(xp)
    neigh = idx[:, :, :KNN].reshape(-1)
    center = jnp.repeat(jnp.arange(bsz * n, dtype=jnp.int32), KNN)
    idx_all = jnp.concatenate([center, neigh])
    nb2 = _sc_gather(xp.reshape(bsz * n, _PAD), idx_all)

    pad16 = lambda w: jnp.pad(w, ((0, 0), (0, _PAD - w.shape[1])))
    w1c = pad16(W1[:, :3])
    w1n = pad16(W1[:, 3:])
    row = lambda v: v.reshape(1, -1)

    base = [w1c, w1n, row(b1)]
    st1 = _run_pass(1, nb2, base, npos)
    l2 = base + [st1, row(g1), row(bb1), W2, row(b2)]
    st2 = _run_pass(2, nb2, l2, npos)
    l3 = l2 + [st2, row(g2), row(bb2), W3, row(b3)]
    st3 = _run_pass(3, nb2, l3, npos)
    l4 = l3 + [st3, row(g3), row(bb3), W4, row(b4)]
    st4, m = _run_pass(4, nb2, l4, npos)
    return _finalize(m, st4, row(g4), row(bb4), bsz, n, npos)


# speedup vs baseline: 6.8370x; 1.1239x over previous
import functools

import jax
import jax.numpy as jnp
from jax import lax
from jax.experimental import pallas as pl
from jax.experimental.pallas import tpu as pltpu
from jax.experimental.pallas import tpu_sc as plsc

KNN = 20
EPS = 1e-5
_PAD = 16
_KPAD = 32
_RB = 512
_PB = 5120
_NSC_WORKERS = 32


def _topk_body(xr, xf, out_ref):
    b = pl.program_id(0)
    n = xf.shape[1]
    xblk = xr[0]
    xfull = xf[0]
    g = lax.dot_general(xblk, xfull, (((1,), (1,)), ((), ())),
                        preferred_element_type=jnp.float32)
    xxr = jnp.sum(xblk * xblk, axis=1, keepdims=True)
    xxf = jnp.sum(xfull * xfull, axis=1, keepdims=True)
    inner = -2.0 * g
    pd = (-xxr) - inner
    pd = pd - xxf.T
    iota = lax.broadcasted_iota(jnp.int32, pd.shape, 1)
    rb = pd.shape[0]
    rowv = pl.program_id(1) * rb + lax.broadcasted_iota(jnp.int32, (rb, 1), 0)
    out_ref[0, :, 0] = rowv[:, 0] + b * n
    pd = jnp.where(iota == rowv, -3e38, pd)
    for r in range(1, KNN):
        m = jnp.max(pd, axis=1, keepdims=True)
        eq = pd == m
        col = jnp.min(jnp.where(eq, iota, n), axis=1)
        out_ref[0, :, r] = col + b * n
        pd = jnp.where(iota == col[:, None], -3e38, pd)


def _topk(xp):
    bsz, n, _ = xp.shape
    rb = min(_RB, n)
    return pl.pallas_call(
        _topk_body,
        grid=(bsz, n // rb),
        in_specs=[
            pl.BlockSpec((1, rb, _PAD), lambda b, r: (b, r, 0)),
            pl.BlockSpec((1, n, _PAD), lambda b, r: (b, 0, 0)),
        ],
        out_specs=pl.BlockSpec((1, rb, _KPAD), lambda b, r: (b, r, 0)),
        out_shape=jax.ShapeDtypeStruct((bsz, n, _KPAD), jnp.int32),
    )(xp, xp)


def _pick_chunk(rows_per_w):
    max_rows = (256 * 1024) // (_PAD * 4)
    ch = rows_per_w
    while ch > max_rows or ch % 8:
        for div in range(2, rows_per_w + 1):
            if rows_per_w % div == 0 and rows_per_w // div <= max_rows:
                ch = rows_per_w // div
                break
        else:
            ch = rows_per_w
            break
        break
    return ch


def _sc_gather(table, idx_all):
    nidx = idx_all.shape[0]
    rows_per_w = nidx // _NSC_WORKERS
    ch = _pick_chunk(rows_per_w)
    chunks = rows_per_w // ch
    mesh = plsc.VectorSubcoreMesh(core_axis_name="c", subcore_axis_name="s")

    @functools.partial(
        pl.kernel,
        out_type=jax.ShapeDtypeStruct((nidx, _PAD), jnp.float32),
        mesh=mesh,
        compiler_params=pltpu.CompilerParams(use_tc_tiling_on_sc=False),
        scratch_types=[
            pltpu.VMEM((ch,), jnp.int32),
            pltpu.VMEM((ch, _PAD), jnp.float32),
            pltpu.SemaphoreType.DMA,
        ],
    )
    def _gather_kernel(table_hbm, idx_hbm, out_hbm, idx_v, rows_v, sem):
        wid = lax.axis_index("s") * 2 + lax.axis_index("c")
        for j in range(chunks):
            base = wid * rows_per_w + j * ch
            pltpu.sync_copy(idx_hbm.at[pl.ds(base, ch)], idx_v)
            pltpu.async_copy(table_hbm.at[idx_v], rows_v, sem).wait()
            pltpu.sync_copy(rows_v, out_hbm.at[pl.ds(base, ch)])

    return _gather_kernel(table, idx_all)


def _affine(s, stats_ref, g_ref, bb_ref, npos):
    mean = stats_ref[0:1, :] / npos
    var = stats_ref[1:2, :] / npos - mean * mean
    return (s - mean) / jnp.sqrt(var + EPS) * g_ref[0:1, :] + bb_ref[0:1, :]


def _acc_stats(stats_ref, s):
    @pl.when(pl.program_id(0) == 0)
    def _():
        stats_ref[...] = jnp.zeros_like(stats_ref)

    stats_ref[0:1, :] += jnp.sum(s, axis=0, keepdims=True)
    stats_ref[1:2, :] += jnp.sum(s * s, axis=0, keepdims=True)


def _mm(a, w):
    return lax.dot_general(a, w, (((1,), (1,)), ((), ())),
                           preferred_element_type=jnp.float32)


def _make_pass_body(stage, npos):
    def body(*refs):
        ctr, nb, w1c, w1n, b1 = refs[:5]
        c = ctr[...]
        s = _mm(c, w1c[...]) + _mm(nb[...] - c, w1n[...]) + b1[0:1, :]
        i = 5
        for params in range(2, stage + 1):
            st, g, bb, w, bvec = refs[i:i + 5]
            i += 5
            h = jnp.maximum(_affine(s, st, g, bb, npos), 0.0)
            s = _mm(h, w[...]) + bvec[0:1, :]
        if stage < 4:
            _acc_stats(refs[-1], s)
        else:
            stats_ref, m_ref = refs[-2], refs[-1]
            _acc_stats(stats_ref, s)
            r = s.shape[0] // KNN
            m_ref[...] = jnp.max(s.reshape(r, KNN, s.shape[1]), axis=1)
    return body


def _run_pass(stage, nb2, wlist, npos):
    nblk = npos // _PB
    cdims = [64, 64, 128, 128]
    cs = cdims[stage - 1]
    full = lambda a: pl.BlockSpec(a.shape, lambda i: tuple(0 for _ in a.shape))
    in_specs = [
        pl.BlockSpec((_PB, _PAD), lambda i: (i, 0)),
        pl.BlockSpec((_PB, _PAD), lambda i: (i + nblk, 0)),
    ] + [full(w) for w in wlist]
    out_specs = pl.BlockSpec((8, cs), lambda i: (0, 0))
    out_shape = jax.ShapeDtypeStruct((8, cs), jnp.float32)
    if stage == 4:
        rpb = _PB // KNN
        out_specs = [out_specs, pl.BlockSpec((rpb, cs), lambda i: (i, 0))]
        out_shape = [out_shape,
                     jax.ShapeDtypeStruct((npos // KNN, cs), jnp.float32)]
    return pl.pallas_call(
        _make_pass_body(stage, float(npos)),
        grid=(nblk,),
        in_specs=in_specs,
        out_specs=out_specs,
        out_shape=out_shape,
    )(nb2, nb2, *wlist)


def _final_body(m_ref, st_ref, g_ref, bb_ref, out_ref, *, npos):
    val = _affine(m_ref[...], st_ref, g_ref, bb_ref, npos)
    out_ref[0] = val.T


def _finalize(m, stats4, g4, bb4, bsz, n, npos):
    c = m.shape[1]
    nj = n // 128
    return pl.pallas_call(
        functools.partial(_final_body, npos=float(npos)),
        grid=(bsz, nj),
        in_specs=[
            pl.BlockSpec((128, c), lambda b, j: (b * nj + j, 0)),
            pl.BlockSpec((8, c), lambda b, j: (0, 0)),
            pl.BlockSpec((1, c), lambda b, j: (0, 0)),
            pl.BlockSpec((1, c), lambda b, j: (0, 0)),
        ],
        out_specs=pl.BlockSpec((1, c, 128), lambda b, j: (b, 0, j)),
        out_shape=jax.ShapeDtypeStruct((bsz, c, n), jnp.float32),
    )(m, stats4, g4, bb4)


def kernel(x, W1, b1, g1, bb1, W2, b2, g2, bb2, W3, b3, g3, bb3,
           W4, b4, g4, bb4):
    bsz, n, _ = x.shape
    npos = bsz * n * KNN
    xp = jnp.pad(x, ((0, 0), (0, 0), (0, _PAD - 3)))
    idx = _topk(xp)
    neigh = idx[:, :, :KNN].reshape(-1)
    center = jnp.repeat(jnp.arange(bsz * n, dtype=jnp.int32), KNN)
    idx_all = jnp.concatenate([center, neigh])
    nb2 = _sc_gather(xp.reshape(bsz * n, _PAD), idx_all)

    pad16 = lambda w: jnp.pad(w, ((0, 0), (0, _PAD - w.shape[1])))
    w1c = pad16(W1[:, :3])
    w1n = pad16(W1[:, 3:])
    row = lambda v: v.reshape(1, -1)

    base = [w1c, w1n, row(b1)]
    st1 = _run_pass(1, nb2, base, npos)
    l2 = base + [st1, row(g1), row(bb1), W2, row(b2)]
    st2 = _run_pass(2, nb2, l2, npos)
    l3 = l2 + [st2, row(g2), row(bb2), W3, row(b3)]
    st3 = _run_pass(3, nb2, l3, npos)
    l4 = l3 + [st3, row(g3), row(bb3), W4, row(b4)]
    st4, m = _run_pass(4, nb2, l4, npos)
    return _finalize(m, st4, row(g4), row(bb4), bsz, n, npos)


# speedup vs baseline: 7.7405x; 1.1322x over previous
import functools

import jax
import jax.numpy as jnp
from jax import lax
from jax.experimental import pallas as pl
from jax.experimental.pallas import tpu as pltpu
from jax.experimental.pallas import tpu_sc as plsc

KNN = 20
EPS = 1e-5
_PAD = 16
_KPAD = 32
_RB = 512
_GB = 4096
_NSC_WORKERS = 32


def _topk_body(xr, xf, out_ref):
    b = pl.program_id(0)
    n = xf.shape[1]
    xblk = xr[0]
    xfull = xf[0]
    g = lax.dot_general(xblk, xfull, (((1,), (1,)), ((), ())),
                        preferred_element_type=jnp.float32)
    xxr = jnp.sum(xblk * xblk, axis=1, keepdims=True)
    xxf = jnp.sum(xfull * xfull, axis=1, keepdims=True)
    inner = -2.0 * g
    pd = (-xxr) - inner
    pd = pd - xxf.T
    iota = lax.broadcasted_iota(jnp.int32, pd.shape, 1)
    rb = pd.shape[0]
    rowv = pl.program_id(1) * rb + lax.broadcasted_iota(jnp.int32, (rb, 1), 0)
    out_ref[0, :, 0] = rowv[:, 0] + b * n
    pd = jnp.where(iota == rowv, -3e38, pd)
    for r in range(1, KNN):
        m = jnp.max(pd, axis=1, keepdims=True)
        eq = pd == m
        col = jnp.min(jnp.where(eq, iota, n), axis=1)
        out_ref[0, :, r] = col + b * n
        pd = jnp.where(iota == col[:, None], -3e38, pd)


def _topk(xp):
    bsz, n, _ = xp.shape
    rb = min(_RB, n)
    return pl.pallas_call(
        _topk_body,
        grid=(bsz, n // rb),
        in_specs=[
            pl.BlockSpec((1, rb, _PAD), lambda b, r: (b, r, 0)),
            pl.BlockSpec((1, n, _PAD), lambda b, r: (b, 0, 0)),
        ],
        out_specs=pl.BlockSpec((1, rb, _KPAD), lambda b, r: (b, r, 0)),
        out_shape=jax.ShapeDtypeStruct((bsz, n, _KPAD), jnp.int32),
    )(xp, xp)


def _pick_chunk(rows_per_w):
    max_rows = (256 * 1024) // (_PAD * 4)
    ch = rows_per_w
    while ch > max_rows or ch % 8:
        for div in range(2, rows_per_w + 1):
            if rows_per_w % div == 0 and rows_per_w // div <= max_rows:
                ch = rows_per_w // div
                break
        else:
            ch = rows_per_w
            break
        break
    return ch


def _sc_gather(table, idx_all):
    nidx = idx_all.shape[0]
    rows_per_w = nidx // _NSC_WORKERS
    ch = _pick_chunk(rows_per_w)
    chunks = rows_per_w // ch
    mesh = plsc.VectorSubcoreMesh(core_axis_name="c", subcore_axis_name="s")

    @functools.partial(
        pl.kernel,
        out_type=jax.ShapeDtypeStruct((nidx, _PAD), jnp.float32),
        mesh=mesh,
        compiler_params=pltpu.CompilerParams(use_tc_tiling_on_sc=False),
        scratch_types=[
            pltpu.VMEM((ch,), jnp.int32),
            pltpu.VMEM((ch, _PAD), jnp.float32),
            pltpu.SemaphoreType.DMA,
        ],
    )
    def _gather_kernel(table_hbm, idx_hbm, out_hbm, idx_v, rows_v, sem):
        wid = lax.axis_index("s") * 2 + lax.axis_index("c")
        for j in range(chunks):
            base = wid * rows_per_w + j * ch
            pltpu.sync_copy(idx_hbm.at[pl.ds(base, ch)], idx_v)
            pltpu.async_copy(table_hbm.at[idx_v], rows_v, sem).wait()
            pltpu.sync_copy(rows_v, out_hbm.at[pl.ds(base, ch)])

    return _gather_kernel(table, idx_all)


def _affine(s, stats_ref, g_ref, bb_ref, npos):
    mean = stats_ref[0:1, :] / npos
    var = stats_ref[1:2, :] / npos - mean * mean
    return (s - mean) / jnp.sqrt(var + EPS) * g_ref[0:1, :] + bb_ref[0:1, :]


def _acc_stats(stats_ref, s):
    @pl.when(pl.program_id(0) == 0)
    def _():
        stats_ref[...] = jnp.zeros_like(stats_ref)

    stats_ref[0:1, :] += jnp.sum(s, axis=0, keepdims=True)
    stats_ref[1:2, :] += jnp.sum(s * s, axis=0, keepdims=True)


def _mm(a, w):
    return lax.dot_general(a, w, (((1,), (1,)), ((), ())),
                           preferred_element_type=jnp.float32)


def _make_pass_body(stage, npos):
    def body(*refs):
        ctr, nb, w1, b1 = refs[:4]
        c = ctr[...]
        x2 = jnp.concatenate([c, nb[...] - c], axis=1)
        s = _mm(x2, w1[...]) + b1[0:1, :]
        i = 4
        for _layer in range(2, stage + 1):
            st, g, bb, w, bvec = refs[i:i + 5]
            i += 5
            h = jnp.maximum(_affine(s, st, g, bb, npos), 0.0)
            s = _mm(h, w[...]) + bvec[0:1, :]
        if stage < 4:
            _acc_stats(refs[-1], s)
        else:
            stats_ref, m_ref = refs[-2], refs[-1]
            k = pl.program_id(1)

            @pl.when((pl.program_id(0) == 0) & (k == 0))
            def _():
                stats_ref[...] = jnp.zeros_like(stats_ref)

            stats_ref[0:1, :] += jnp.sum(s, axis=0, keepdims=True)
            stats_ref[1:2, :] += jnp.sum(s * s, axis=0, keepdims=True)

            @pl.when(k == 0)
            def _():
                m_ref[...] = s

            @pl.when(k != 0)
            def _():
                m_ref[...] = jnp.maximum(m_ref[...], s)
    return body


def _run_pass(stage, table, nbk, wlist, npos):
    npts = npos // KNN
    gb_sz = min(_GB, npts)
    gpk = npts // gb_sz
    cdims = [64, 64, 128, 128]
    cs = cdims[stage - 1]
    full = lambda a: pl.BlockSpec(a.shape, lambda *i: tuple(0 for _ in a.shape))
    if stage < 4:
        grid = (npos // gb_sz,)
        ctr_spec = pl.BlockSpec((gb_sz, _PAD), lambda i: (i % gpk, 0))
        nb_spec = pl.BlockSpec((gb_sz, _PAD), lambda i: (i, 0))
        out_specs = pl.BlockSpec((8, cs), lambda i: (0, 0))
        out_shape = jax.ShapeDtypeStruct((8, cs), jnp.float32)
    else:
        grid = (gpk, KNN)
        ctr_spec = pl.BlockSpec((gb_sz, _PAD), lambda gb, k: (gb, 0))
        nb_spec = pl.BlockSpec((gb_sz, _PAD), lambda gb, k: (k * gpk + gb, 0))
        out_specs = [pl.BlockSpec((8, cs), lambda gb, k: (0, 0)),
                     pl.BlockSpec((gb_sz, cs), lambda gb, k: (gb, 0))]
        out_shape = [jax.ShapeDtypeStruct((8, cs), jnp.float32),
                     jax.ShapeDtypeStruct((npts, cs), jnp.float32)]
    return pl.pallas_call(
        _make_pass_body(stage, float(npos)),
        grid=grid,
        in_specs=[ctr_spec, nb_spec] + [full(w) for w in wlist],
        out_specs=out_specs,
        out_shape=out_shape,
    )(table, nbk, *wlist)


def _final_body(m_ref, st_ref, g_ref, bb_ref, out_ref, *, npos):
    val = _affine(m_ref[...], st_ref, g_ref, bb_ref, npos)
    out_ref[0] = val.T


def _finalize(m, stats4, g4, bb4, bsz, n, npos):
    c = m.shape[1]
    nj = n // 128
    return pl.pallas_call(
        functools.partial(_final_body, npos=float(npos)),
        grid=(bsz, nj),
        in_specs=[
            pl.BlockSpec((128, c), lambda b, j: (b * nj + j, 0)),
            pl.BlockSpec((8, c), lambda b, j: (0, 0)),
            pl.BlockSpec((1, c), lambda b, j: (0, 0)),
            pl.BlockSpec((1, c), lambda b, j: (0, 0)),
        ],
        out_specs=pl.BlockSpec((1, c, 128), lambda b, j: (b, 0, j)),
        out_shape=jax.ShapeDtypeStruct((bsz, c, n), jnp.float32),
    )(m, stats4, g4, bb4)


def kernel(x, W1, b1, g1, bb1, W2, b2, g2, bb2, W3, b3, g3, bb3,
           W4, b4, g4, bb4):
    bsz, n, _ = x.shape
    npos = bsz * n * KNN
    xp = jnp.pad(x, ((0, 0), (0, 0), (0, _PAD - 3)))
    idx = _topk(xp)
    neigh = idx[:, :, :KNN].reshape(bsz * n, KNN).T.reshape(-1)
    table = xp.reshape(bsz * n, _PAD)
    nbk = _sc_gather(table, neigh)

    pad16 = lambda w: jnp.pad(w, ((0, 0), (0, _PAD - w.shape[1])))
    w1cat = jnp.concatenate([pad16(W1[:, :3]), pad16(W1[:, 3:])], axis=1)
    row = lambda v: v.reshape(1, -1)

    base = [w1cat, row(b1)]
    st1 = _run_pass(1, table, nbk, base, npos)
    l2 = base + [st1, row(g1), row(bb1), W2, row(b2)]
    st2 = _run_pass(2, table, nbk, l2, npos)
    l3 = l2 + [st2, row(g2), row(bb2), W3, row(b3)]
    st3 = _run_pass(3, table, nbk, l3, npos)
    l4 = l3 + [st3, row(g3), row(bb3), W4, row(b4)]
    st4, m = _run_pass(4, table, nbk, l4, npos)
    return _finalize(m, st4, row(g4), row(bb4), bsz, n, npos)
